# baseline (speedup 1.0000x reference)
import functools

import jax
import jax.numpy as jnp
from jax import lax
from jax.experimental import pallas as pl
from jax.experimental.pallas import tpu as pltpu
from jax.experimental.pallas import tpu_sc as plsc

_EPS = 1e-12
_L = 16


def _rsqrt(x):
    y = lax.bitcast_convert_type(
        jnp.full(x.shape, 0x5F3759DF, jnp.int32)
        - (lax.bitcast_convert_type(x, jnp.int32) >> 1),
        jnp.float32,
    )
    for _ in range(3):
        y = y * (1.5 - 0.5 * x * y * y)
    return y


def kernel(input_ids, item_table, pos_table, ln_gamma, ln_beta):
    B, S = input_ids.shape
    V, H = item_table.shape
    CH = 2 * S
    NB = CH // _L
    half = S // 2
    npairs = B // 2
    ids = input_ids.astype(jnp.int32).reshape(npairs, 4, half)

    info = plsc.get_sparse_core_info()
    NC, NS = info.num_cores, info.num_subcores
    NW = NC * NS
    pairs_per_w = npairs // NW

    mesh = plsc.VectorSubcoreMesh(core_axis_name="c", subcore_axis_name="s")

    @functools.partial(
        pl.kernel,
        out_type=jax.ShapeDtypeStruct((npairs, CH, H), jnp.float32),
        mesh=mesh,
        compiler_params=pltpu.CompilerParams(
            needs_layout_passes=False, use_tc_tiling_on_sc=False),
        scratch_types=[
            pltpu.VMEM((4, half), jnp.int32),
            pltpu.VMEM((CH, H), jnp.float32),
            pltpu.VMEM((S, H), jnp.float32),
            pltpu.VMEM((H,), jnp.float32),
            pltpu.VMEM((H,), jnp.float32),
            pltpu.SemaphoreType.DMA,
        ],
    )
    def emb_ln(ids_hbm, table_hbm, pos_hbm, gamma_hbm, beta_hbm, out_hbm,
               idx_v, rows_v, pos_v, gamma_v, beta_v, sem):
        wid = lax.axis_index("c") * NS + lax.axis_index("s")
        p0 = wid * pairs_per_w

        pltpu.sync_copy(pos_hbm.at[pl.ds(0, S)], pos_v)
        pltpu.sync_copy(gamma_hbm, gamma_v)
        pltpu.sync_copy(beta_hbm, beta_v)
        gv = [gamma_v[pl.ds(k * _L, _L)] for k in range(H // _L)]
        bv = [beta_v[pl.ds(k * _L, _L)] for k in range(H // _L)]
        iota = lax.iota(jnp.int32, _L)

        def per_chunk(pi, _):
            p = p0 + pi
            pltpu.sync_copy(ids_hbm.at[p], idx_v)
            copies = [
                pltpu.async_copy(
                    table_hbm.at[idx_v.at[j]],
                    rows_v.at[pl.ds(j * half, half)], sem)
                for j in range(4)
            ]
            for c in copies:
                c.wait()

            def per_block(bi, _):
                rvec = bi * _L + iota
                prvec = jnp.where(rvec >= S, rvec - S, rvec)
                s_acc = jnp.zeros((_L,), jnp.float32)
                q_acc = jnp.zeros((_L,), jnp.float32)
                for j in range(H):
                    cvec = jnp.full((_L,), j, jnp.int32)
                    t = (plsc.load_gather(rows_v, [rvec, cvec])
                         + plsc.load_gather(pos_v, [prvec, cvec]))
                    s_acc = s_acc + t
                    q_acc = q_acc + t * t
                    plsc.store_scatter(rows_v, [rvec, cvec], t)
                mean = s_acc * (1.0 / H)
                var = q_acc * (1.0 / H) - mean * mean
                r = _rsqrt(var + _EPS)
                for j in range(H):
                    cvec = jnp.full((_L,), j, jnp.int32)
                    t = plsc.load_gather(rows_v, [rvec, cvec])
                    y = ((t - mean) * r * gv[j // _L][j % _L]
                         + bv[j // _L][j % _L])
                    plsc.store_scatter(rows_v, [rvec, cvec], y)
                return ()

            lax.fori_loop(0, NB, per_block, ())
            pltpu.sync_copy(rows_v, out_hbm.at[p])
            return ()

        lax.fori_loop(0, pairs_per_w, per_chunk, ())

    out = emb_ln(ids, item_table, pos_table, ln_gamma, ln_beta)
    return out.reshape(B, S, H)


# speedup vs baseline: 2.3091x; 2.3091x over previous
import functools

import jax
import jax.numpy as jnp
from jax import lax
from jax.experimental import pallas as pl
from jax.experimental.pallas import tpu as pltpu
from jax.experimental.pallas import tpu_sc as plsc

_EPS = 1e-12
_L = 16


def _rsqrt(x):
    y = lax.bitcast_convert_type(
        0x5F3759DF - (lax.bitcast_convert_type(x, jnp.int32) >> 1),
        jnp.float32,
    )
    for _ in range(3):
        y = y * (1.5 - 0.5 * x * y * y)
    return y


def kernel(input_ids, item_table, pos_table, ln_gamma, ln_beta):
    B, S = input_ids.shape
    V, H = item_table.shape
    half = S // 2
    K = H // _L
    ids = input_ids.astype(jnp.int32).reshape(B, 2, half)

    info = plsc.get_sparse_core_info()
    NC, NS = info.num_cores, info.num_subcores
    NW = NC * NS
    seq_per_w = B // NW

    mesh = plsc.VectorSubcoreMesh(core_axis_name="c", subcore_axis_name="s")

    @functools.partial(
        pl.kernel,
        out_type=jax.ShapeDtypeStruct((B, S, H), jnp.float32),
        mesh=mesh,
        compiler_params=pltpu.CompilerParams(
            needs_layout_passes=False, use_tc_tiling_on_sc=False),
        scratch_types=[
            pltpu.VMEM((2, half), jnp.int32),
            pltpu.VMEM((S, H), jnp.float32),
            pltpu.VMEM((S, H), jnp.float32),
            pltpu.VMEM((H,), jnp.float32),
            pltpu.VMEM((H,), jnp.float32),
            pltpu.SemaphoreType.DMA,
        ],
    )
    def emb_ln(ids_hbm, table_hbm, pos_hbm, gamma_hbm, beta_hbm, out_hbm,
               idx_v, rows_v, pos_v, gamma_v, beta_v, sem):
        wid = lax.axis_index("c") * NS + lax.axis_index("s")
        q0 = wid * seq_per_w

        pltpu.sync_copy(pos_hbm.at[pl.ds(0, S)], pos_v)
        pltpu.sync_copy(gamma_hbm, gamma_v)
        pltpu.sync_copy(beta_hbm, beta_v)
        gv = [gamma_v[pl.ds(k * _L, _L)] for k in range(K)]
        bv = [beta_v[pl.ds(k * _L, _L)] for k in range(K)]

        def per_seq(qi, _):
            q = q0 + qi
            pltpu.sync_copy(ids_hbm.at[q], idx_v)
            c0 = pltpu.async_copy(
                table_hbm.at[idx_v.at[0]], rows_v.at[pl.ds(0, half)], sem)
            c1 = pltpu.async_copy(
                table_hbm.at[idx_v.at[1]], rows_v.at[pl.ds(half, half)], sem)
            c0.wait()
            c1.wait()

            def per_row(i, _):
                x = [rows_v[i, pl.ds(k * _L, _L)] + pos_v[i, pl.ds(k * _L, _L)]
                     for k in range(K)]
                tot = jnp.sum((x[0] + x[1]) + (x[2] + x[3]))
                mean = tot * (1.0 / H)
                d = [xk - mean for xk in x]
                sq = (d[0] * d[0] + d[1] * d[1]) + (d[2] * d[2] + d[3] * d[3])
                var = jnp.sum(sq) * (1.0 / H)
                r = _rsqrt(var + _EPS)
                for k in range(K):
                    rows_v[i, pl.ds(k * _L, _L)] = d[k] * r * gv[k] + bv[k]
                return ()

            lax.fori_loop(0, S, per_row, (), unroll=2)
            pltpu.sync_copy(rows_v, out_hbm.at[q])
            return ()

        lax.fori_loop(0, seq_per_w, per_seq, ())

    out = emb_ln(ids, item_table, pos_table, ln_gamma, ln_beta)
    return out
